# baseline (speedup 1.0000x reference)
import functools

import jax
import jax.numpy as jnp
from jax.experimental import pallas as pl
from jax.experimental.pallas import tpu as pltpu

BM = 4096
BK = 256


def _fused_body(att_ref, x_ref, w_ref, a_ref, b_ref, o_ref, s_ref,
                *, num_rel, num_k):
    r = pl.program_id(1)
    k = pl.program_id(2)

    @pl.when((r == 0) & (k == 0))
    def _compute_support():
        m = att_ref[0]
        for j in range(1, num_rel):
            m = jnp.maximum(m, att_ref[j])
        denom = jnp.exp(att_ref[0] - m)
        for j in range(1, num_rel):
            denom = denom + jnp.exp(att_ref[j] - m)
        x = x_ref[...]
        for j in range(num_rel):
            att_j = jnp.exp(att_ref[j] - m) / denom
            s_ref[j] = (jnp.dot(x, w_ref[j], preferred_element_type=jnp.float32)
                        * att_j).astype(jnp.bfloat16)
        o_ref[...] = jnp.broadcast_to(b_ref[...], o_ref.shape)

    o_ref[...] += jnp.dot(a_ref[0].astype(jnp.bfloat16),
                          s_ref[r, pl.ds(k * BK, BK), :],
                          preferred_element_type=jnp.float32)


def kernel(input, adjs, adj_weight, attention, bias):
    num_rel, n, _ = adjs.shape
    d_in = input.shape[1]
    d_out = adj_weight.shape[2]
    num_k = n // BK

    out = pl.pallas_call(
        functools.partial(_fused_body, num_rel=num_rel, num_k=num_k),
        grid=(n // BM, num_rel, num_k),
        in_specs=[
            pl.BlockSpec(memory_space=pltpu.SMEM),
            pl.BlockSpec((n, d_in), lambda i, r, k: (0, 0)),
            pl.BlockSpec((num_rel, d_in, d_out), lambda i, r, k: (0, 0, 0)),
            pl.BlockSpec((1, BM, BK), lambda i, r, k: (r, i, k)),
            pl.BlockSpec((1, d_out), lambda i, r, k: (0, 0)),
        ],
        out_specs=pl.BlockSpec((BM, d_out), lambda i, r, k: (i, 0)),
        out_shape=jax.ShapeDtypeStruct((n, d_out), jnp.float32),
        scratch_shapes=[pltpu.VMEM((num_rel, n, d_out), jnp.bfloat16)],
        compiler_params=pltpu.CompilerParams(
            dimension_semantics=("parallel", "arbitrary", "arbitrary"),
        ),
    )(attention, input, adj_weight, adjs, bias.reshape(1, d_out))
    return out


# speedup vs baseline: 1.1740x; 1.1740x over previous
import functools

import jax
import jax.numpy as jnp
from jax.experimental import pallas as pl
from jax.experimental.pallas import tpu as pltpu

BM = 4096
BK = 512


def _fused_body(att_ref, x_ref, w_ref, a0_ref, a1_ref, b_ref, o_ref, s_ref,
                *, num_rel, num_k2):
    r = pl.program_id(1)
    k = pl.program_id(2)

    @pl.when((r == 0) & (k == 0))
    def _compute_support():
        m = att_ref[0]
        for j in range(1, num_rel):
            m = jnp.maximum(m, att_ref[j])
        denom = jnp.exp(att_ref[0] - m)
        for j in range(1, num_rel):
            denom = denom + jnp.exp(att_ref[j] - m)
        x = x_ref[...]
        for j in range(num_rel):
            att_j = jnp.exp(att_ref[j] - m) / denom
            s_ref[j] = (jnp.dot(x, w_ref[j], preferred_element_type=jnp.float32)
                        * att_j).astype(jnp.bfloat16)
        o_ref[...] = jnp.broadcast_to(b_ref[...], o_ref.shape)

    acc = jnp.dot(a0_ref[0].astype(jnp.bfloat16),
                  s_ref[r, pl.ds((2 * k) * BK, BK), :],
                  preferred_element_type=jnp.float32)
    acc += jnp.dot(a1_ref[0].astype(jnp.bfloat16),
                   s_ref[r, pl.ds((2 * k + 1) * BK, BK), :],
                   preferred_element_type=jnp.float32)
    o_ref[...] += acc


def kernel(input, adjs, adj_weight, attention, bias):
    num_rel, n, _ = adjs.shape
    d_in = input.shape[1]
    d_out = adj_weight.shape[2]
    num_k2 = n // (2 * BK)

    out = pl.pallas_call(
        functools.partial(_fused_body, num_rel=num_rel, num_k2=num_k2),
        grid=(n // BM, num_rel, num_k2),
        in_specs=[
            pl.BlockSpec(memory_space=pltpu.SMEM),
            pl.BlockSpec((n, d_in), lambda i, r, k: (0, 0)),
            pl.BlockSpec((num_rel, d_in, d_out), lambda i, r, k: (0, 0, 0)),
            pl.BlockSpec((1, BM, BK), lambda i, r, k: (r, i, 2 * k)),
            pl.BlockSpec((1, BM, BK), lambda i, r, k: (r, i, 2 * k + 1)),
            pl.BlockSpec((1, d_out), lambda i, r, k: (0, 0)),
        ],
        out_specs=pl.BlockSpec((BM, d_out), lambda i, r, k: (i, 0)),
        out_shape=jax.ShapeDtypeStruct((n, d_out), jnp.float32),
        scratch_shapes=[pltpu.VMEM((num_rel, n, d_out), jnp.bfloat16)],
        compiler_params=pltpu.CompilerParams(
            dimension_semantics=("parallel", "arbitrary", "arbitrary"),
        ),
    )(attention, input, adj_weight, adjs, adjs, bias.reshape(1, d_out))
    return out


# speedup vs baseline: 1.2309x; 1.0484x over previous
import functools

import jax
import jax.numpy as jnp
from jax.experimental import pallas as pl
from jax.experimental.pallas import tpu as pltpu

BM = 4096
BK = 512


def _softmax_weight(att_ref, j, num_rel):
    m = att_ref[0]
    for t in range(1, num_rel):
        m = jnp.maximum(m, att_ref[t])
    denom = jnp.exp(att_ref[0] - m)
    for t in range(1, num_rel):
        denom = denom + jnp.exp(att_ref[t] - m)
    return jnp.exp(att_ref[j] - m) / denom


def _fused_body(att_ref, x_ref, w_ref, a_ref, b_ref, o_ref, s_ref,
                *, num_rel, num_k):
    r = pl.program_id(1)
    k = pl.program_id(2)

    @pl.when((r == 0) & (k == 0))
    def _support_first():
        att_0 = _softmax_weight(att_ref, 0, num_rel)
        s_ref[0] = (jnp.dot(x_ref[...], w_ref[0],
                            preferred_element_type=jnp.float32)
                    * att_0).astype(jnp.bfloat16)
        o_ref[...] = jnp.broadcast_to(b_ref[...], o_ref.shape)

    @pl.when((r == 0) & (k == 1))
    def _support_rest():
        x = x_ref[...]
        for j in range(1, num_rel):
            att_j = _softmax_weight(att_ref, j, num_rel)
            s_ref[j] = (jnp.dot(x, w_ref[j], preferred_element_type=jnp.float32)
                        * att_j).astype(jnp.bfloat16)

    o_ref[...] += jnp.dot(a_ref[0].astype(jnp.bfloat16),
                          s_ref[r, pl.ds(k * BK, BK), :],
                          preferred_element_type=jnp.float32)


def kernel(input, adjs, adj_weight, attention, bias):
    num_rel, n, _ = adjs.shape
    d_in = input.shape[1]
    d_out = adj_weight.shape[2]
    num_k = n // BK

    out = pl.pallas_call(
        functools.partial(_fused_body, num_rel=num_rel, num_k=num_k),
        grid=(n // BM, num_rel, num_k),
        in_specs=[
            pl.BlockSpec(memory_space=pltpu.SMEM),
            pl.BlockSpec((n, d_in), lambda i, r, k: (0, 0)),
            pl.BlockSpec((num_rel, d_in, d_out), lambda i, r, k: (0, 0, 0)),
            pl.BlockSpec((1, BM, BK), lambda i, r, k: (r, i, k)),
            pl.BlockSpec((1, d_out), lambda i, r, k: (0, 0)),
        ],
        out_specs=pl.BlockSpec((BM, d_out), lambda i, r, k: (i, 0)),
        out_shape=jax.ShapeDtypeStruct((n, d_out), jnp.float32),
        scratch_shapes=[pltpu.VMEM((num_rel, n, d_out), jnp.bfloat16)],
        compiler_params=pltpu.CompilerParams(
            dimension_semantics=("parallel", "arbitrary", "arbitrary"),
        ),
    )(attention, input, adj_weight, adjs, bias.reshape(1, d_out))
    return out
